# baseline (speedup 1.0000x reference)
import functools

import jax
import jax.numpy as jnp
from jax import lax
from jax.experimental import pallas as pl
from jax.experimental.pallas import tpu as pltpu
from jax.experimental.pallas import tpu_sc as plsc

N_ROWS = 1_000_000
N_COLS = 10
N_BINS = 15
NC, NS, L = 2, 16, 16
NW = NC * NS
CHUNK = 1792
N_CHUNKS = N_ROWS // CHUNK
SC_ROWS = N_CHUNKS * CHUNK
TAIL = N_ROWS - SC_ROWS
GROUPS = CHUNK // L


def _ece_partials(lt, bt):
    mesh = plsc.VectorSubcoreMesh(
        core_axis_name="c", subcore_axis_name="s",
        num_cores=NC, num_subcores=NS)

    buf_t = pltpu.VMEM((N_COLS, CHUNK), jnp.float32)

    @functools.partial(
        pl.kernel,
        out_type=jax.ShapeDtypeStruct((NW, 48, L), jnp.float32),
        mesh=mesh,
        scratch_types=[
            buf_t, buf_t,
            buf_t, buf_t,
            pltpu.VMEM((48, L), jnp.float32),
            pltpu.SemaphoreType.DMA,
            pltpu.SemaphoreType.DMA,
            pltpu.SemaphoreType.DMA,
            pltpu.SemaphoreType.DMA,
        ],
        compiler_params=pltpu.CompilerParams(needs_layout_passes=False),
    )
    def sc_kernel(lt_hbm, bt_hbm, out_hbm,
                  lbuf0, lbuf1, bbuf0, bbuf1, part,
                  sl0, sl1, sb0, sb1):
        wid = lax.axis_index("s") * NC + lax.axis_index("c")

        zeros16 = jnp.zeros((L,), jnp.float32)
        for r in range(48):
            part[r, :] = zeros16

        lane = lax.broadcasted_iota(jnp.int32, (L,), 0)
        ones_f = jnp.full((L,), 1.0, jnp.float32)
        fifteen = jnp.full((L,), float(N_BINS), jnp.float32)

        def issue(c, lb, bb, sl, sb):
            base = pl.multiple_of(c * CHUNK, 128)
            for j in range(N_COLS):
                pltpu.make_async_copy(
                    lt_hbm.at[j, pl.ds(base, CHUNK)], lb.at[j], sl).start()
                pltpu.make_async_copy(
                    bt_hbm.at[j, pl.ds(base, CHUNK)], bb.at[j], sb).start()

        def wait(lb, bb, sl, sb):
            for j in range(N_COLS):
                pltpu.make_async_copy(
                    lt_hbm.at[j, pl.ds(0, CHUNK)], lb.at[j], sl).wait()
                pltpu.make_async_copy(
                    bt_hbm.at[j, pl.ds(0, CHUNK)], bb.at[j], sb).wait()

        def compute(lb, bb):
            @plsc.parallel_loop(0, GROUPS, unroll=4)
            def do_group(g):
                s = g * L
                l0 = lb[0, pl.ds(s, L)]
                m = l0
                still_neg = l0 < 0.0
                lead = jnp.where(still_neg, 1, 0).astype(jnp.int32)
                for j in range(1, N_COLS):
                    lj = lb[j, pl.ds(s, L)]
                    m = jnp.maximum(m, lj)
                    still_neg = jnp.logical_and(still_neg, lj < 0.0)
                    lead = lead + jnp.where(still_neg, 1, 0).astype(jnp.int32)
                pred_idx = jnp.where(lead == N_COLS, 0, lead)

                b0 = bb[0, pl.ds(s, L)]
                best = b0
                lidx = jnp.zeros((L,), jnp.int32)
                for j in range(1, N_COLS):
                    bj = bb[j, pl.ds(s, L)]
                    gt = bj > best
                    best = jnp.maximum(best, bj)
                    lidx = jnp.where(gt, j, lidx)

                acc = jnp.where(pred_idx == lidx, 1.0, 0.0).astype(jnp.float32)
                conf = ones_f / (ones_f + jnp.exp(-m))
                bin_i = (conf * fifteen).astype(jnp.int32)
                bin_i = jnp.minimum(jnp.maximum(bin_i, 0), N_BINS - 1)

                valid = conf > 0.0
                plsc.addupdate_scatter(part, [bin_i, lane], ones_f,
                                       mask=valid)
                plsc.addupdate_scatter(part, [bin_i + 16, lane], acc,
                                       mask=valid)
                plsc.addupdate_scatter(part, [bin_i + 32, lane], conf,
                                       mask=valid)

        nk = (N_CHUNKS - wid + NW - 1) // NW
        issue(wid, lbuf0, bbuf0, sl0, sb0)

        def body(k, _):
            nxt = wid + (k + 1) * NW
            even = (k % 2) == 0
            has_next = nxt < N_CHUNKS

            @pl.when(jnp.logical_and(has_next, even))
            def _():
                issue(nxt, lbuf1, bbuf1, sl1, sb1)

            @pl.when(jnp.logical_and(has_next, jnp.logical_not(even)))
            def _():
                issue(nxt, lbuf0, bbuf0, sl0, sb0)

            @pl.when(even)
            def _():
                wait(lbuf0, bbuf0, sl0, sb0)
                compute(lbuf0, bbuf0)

            @pl.when(jnp.logical_not(even))
            def _():
                wait(lbuf1, bbuf1, sl1, sb1)
                compute(lbuf1, bbuf1)

            return 0

        lax.fori_loop(0, nk, body, 0)
        pltpu.sync_copy(part, out_hbm.at[wid])

    return sc_kernel(lt, bt)


def _combine(partials, ltail, btail):
    def tc_kernel(p_ref, lt_ref, bt_ref, o_ref):
        x = p_ref[...]
        tot = jnp.sum(x, axis=(0, 2))
        cnt = tot[0:16]
        acc_s = tot[16:32]
        conf_s = tot[32:48]

        l = lt_ref[...]
        b = bt_ref[...]
        m = jnp.max(l, axis=1)
        nn = (l >= 0.0).astype(jnp.float32)
        pred = jnp.argmax(nn, axis=1)
        lidx = jnp.argmax(b, axis=1)
        accv = (pred == lidx).astype(jnp.float32)
        conf = 1.0 / (1.0 + jnp.exp(-m))
        bin_i = (conf * float(N_BINS)).astype(jnp.int32)
        bin_i = jnp.minimum(jnp.maximum(bin_i, 0), N_BINS - 1)
        validf = (conf > 0.0).astype(jnp.float32)
        bins16 = lax.broadcasted_iota(jnp.int32, (16,), 0)
        onehot = (bin_i[:, None] == bins16[None, :]).astype(jnp.float32)
        onehot = onehot * validf[:, None]
        cnt = cnt + jnp.sum(onehot, axis=0)
        acc_s = acc_s + jnp.sum(onehot * accv[:, None], axis=0)
        conf_s = conf_s + jnp.sum(onehot * conf[:, None], axis=0)

        prop = cnt * (1.0 / N_ROWS)
        safe = jnp.maximum(cnt, 1.0)
        contrib = jnp.abs(conf_s / safe - acc_s / safe) * prop
        contrib = jnp.where(cnt > 0.0, contrib, 0.0)
        o_ref[0, 0] = jnp.sum(contrib)

    out = pl.pallas_call(
        tc_kernel,
        out_shape=jax.ShapeDtypeStruct((1, 1), jnp.float32),
        in_specs=[
            pl.BlockSpec(memory_space=pltpu.VMEM),
            pl.BlockSpec(memory_space=pltpu.VMEM),
            pl.BlockSpec(memory_space=pltpu.VMEM),
        ],
        out_specs=pl.BlockSpec(memory_space=pltpu.SMEM),
    )(partials, ltail, btail)
    return out.reshape((1,))


@jax.jit
def kernel(logits, labels):
    lt = logits.T
    bt = labels.T
    partials = _ece_partials(lt, bt)
    ltail = logits[SC_ROWS:]
    btail = labels[SC_ROWS:]
    return _combine(partials, ltail, btail)


# speedup vs baseline: 1.1140x; 1.1140x over previous
import functools

import jax
import jax.numpy as jnp
from jax import lax
from jax.experimental import pallas as pl
from jax.experimental.pallas import tpu as pltpu
from jax.experimental.pallas import tpu_sc as plsc

N_ROWS = 1_000_000
N_COLS = 10
N_BINS = 15
NC, NS, L = 2, 16, 16
NW = NC * NS
CHUNK = 1792
N_CHUNKS = N_ROWS // CHUNK
SC_ROWS = N_CHUNKS * CHUNK
TAIL = N_ROWS - SC_ROWS
GROUPS = CHUNK // L


def _ece_partials(lt, bt):
    mesh = plsc.VectorSubcoreMesh(
        core_axis_name="c", subcore_axis_name="s",
        num_cores=NC, num_subcores=NS)

    buf_t = pltpu.VMEM((N_COLS, CHUNK), jnp.float32)

    @functools.partial(
        pl.kernel,
        out_type=jax.ShapeDtypeStruct((NW, 48, L), jnp.float32),
        mesh=mesh,
        scratch_types=[
            buf_t, buf_t,
            buf_t, buf_t,
            pltpu.VMEM((48, L), jnp.float32),
            pltpu.SemaphoreType.DMA,
            pltpu.SemaphoreType.DMA,
            pltpu.SemaphoreType.DMA,
            pltpu.SemaphoreType.DMA,
        ],
        compiler_params=pltpu.CompilerParams(needs_layout_passes=False),
    )
    def sc_kernel(lt_hbm, bt_hbm, out_hbm,
                  lbuf0, lbuf1, bbuf0, bbuf1, part,
                  sl0, sl1, sb0, sb1):
        wid = lax.axis_index("s") * NC + lax.axis_index("c")

        zeros16 = jnp.zeros((L,), jnp.float32)
        for r in range(48):
            part[r, :] = zeros16

        lane = lax.broadcasted_iota(jnp.int32, (L,), 0)
        ones_f = jnp.full((L,), 1.0, jnp.float32)
        fifteen = jnp.full((L,), float(N_BINS), jnp.float32)

        def issue(c, lb, bb, sl, sb):
            base = pl.multiple_of(c * CHUNK, 128)
            for j in range(N_COLS):
                pltpu.make_async_copy(
                    lt_hbm.at[j, pl.ds(base, CHUNK)], lb.at[j], sl).start()
                pltpu.make_async_copy(
                    bt_hbm.at[j, pl.ds(base, CHUNK)], bb.at[j], sb).start()

        def wait(lb, bb, sl, sb):
            for j in range(N_COLS):
                pltpu.make_async_copy(
                    lt_hbm.at[j, pl.ds(0, CHUNK)], lb.at[j], sl).wait()
                pltpu.make_async_copy(
                    bt_hbm.at[j, pl.ds(0, CHUNK)], bb.at[j], sb).wait()

        def compute(lb, bb):
            @plsc.parallel_loop(0, GROUPS, unroll=2)
            def do_group(g):
                s = g * L
                l0 = lb[0, pl.ds(s, L)]
                m = l0
                still_neg = l0 < 0.0
                lead = jnp.where(still_neg, 1, 0).astype(jnp.int32)
                for j in range(1, N_COLS):
                    lj = lb[j, pl.ds(s, L)]
                    m = jnp.maximum(m, lj)
                    still_neg = jnp.logical_and(still_neg, lj < 0.0)
                    lead = lead + jnp.where(still_neg, 1, 0).astype(jnp.int32)
                pred_idx = jnp.where(lead == N_COLS, 0, lead)

                b0 = bb[0, pl.ds(s, L)]
                best = b0
                lidx = jnp.zeros((L,), jnp.int32)
                for j in range(1, N_COLS):
                    bj = bb[j, pl.ds(s, L)]
                    gt = bj > best
                    best = jnp.maximum(best, bj)
                    lidx = jnp.where(gt, j, lidx)

                acc = jnp.where(pred_idx == lidx, 1.0, 0.0).astype(jnp.float32)
                conf = ones_f / (ones_f + jnp.exp(-m))
                bin_i = (conf * fifteen).astype(jnp.int32)
                bin_i = jnp.minimum(jnp.maximum(bin_i, 0), N_BINS - 1)

                valid = conf > 0.0
                plsc.addupdate_scatter(part, [bin_i, lane], ones_f,
                                       mask=valid)
                plsc.addupdate_scatter(part, [bin_i + 16, lane], acc,
                                       mask=valid)
                plsc.addupdate_scatter(part, [bin_i + 32, lane], conf,
                                       mask=valid)

        nk = (N_CHUNKS - wid + NW - 1) // NW
        issue(wid, lbuf0, bbuf0, sl0, sb0)

        def body(k, _):
            nxt = wid + (k + 1) * NW
            even = (k % 2) == 0
            has_next = nxt < N_CHUNKS

            @pl.when(jnp.logical_and(has_next, even))
            def _():
                issue(nxt, lbuf1, bbuf1, sl1, sb1)

            @pl.when(jnp.logical_and(has_next, jnp.logical_not(even)))
            def _():
                issue(nxt, lbuf0, bbuf0, sl0, sb0)

            @pl.when(even)
            def _():
                wait(lbuf0, bbuf0, sl0, sb0)
                compute(lbuf0, bbuf0)

            @pl.when(jnp.logical_not(even))
            def _():
                wait(lbuf1, bbuf1, sl1, sb1)
                compute(lbuf1, bbuf1)

            return 0

        lax.fori_loop(0, nk, body, 0)
        pltpu.sync_copy(part, out_hbm.at[wid])

    return sc_kernel(lt, bt)


def _combine(partials, ltail, btail):
    def tc_kernel(p_ref, lt_ref, bt_ref, o_ref):
        x = p_ref[...]
        tot = jnp.sum(x, axis=(0, 2))
        cnt = tot[0:16]
        acc_s = tot[16:32]
        conf_s = tot[32:48]

        l = lt_ref[...]
        b = bt_ref[...]
        m = jnp.max(l, axis=1)
        nn = (l >= 0.0).astype(jnp.float32)
        pred = jnp.argmax(nn, axis=1)
        lidx = jnp.argmax(b, axis=1)
        accv = (pred == lidx).astype(jnp.float32)
        conf = 1.0 / (1.0 + jnp.exp(-m))
        bin_i = (conf * float(N_BINS)).astype(jnp.int32)
        bin_i = jnp.minimum(jnp.maximum(bin_i, 0), N_BINS - 1)
        validf = (conf > 0.0).astype(jnp.float32)
        bins16 = lax.broadcasted_iota(jnp.int32, (16,), 0)
        onehot = (bin_i[:, None] == bins16[None, :]).astype(jnp.float32)
        onehot = onehot * validf[:, None]
        cnt = cnt + jnp.sum(onehot, axis=0)
        acc_s = acc_s + jnp.sum(onehot * accv[:, None], axis=0)
        conf_s = conf_s + jnp.sum(onehot * conf[:, None], axis=0)

        prop = cnt * (1.0 / N_ROWS)
        safe = jnp.maximum(cnt, 1.0)
        contrib = jnp.abs(conf_s / safe - acc_s / safe) * prop
        contrib = jnp.where(cnt > 0.0, contrib, 0.0)
        o_ref[0, 0] = jnp.sum(contrib)

    out = pl.pallas_call(
        tc_kernel,
        out_shape=jax.ShapeDtypeStruct((1, 1), jnp.float32),
        in_specs=[
            pl.BlockSpec(memory_space=pltpu.VMEM),
            pl.BlockSpec(memory_space=pltpu.VMEM),
            pl.BlockSpec(memory_space=pltpu.VMEM),
        ],
        out_specs=pl.BlockSpec(memory_space=pltpu.SMEM),
    )(partials, ltail, btail)
    return out.reshape((1,))


@jax.jit
def kernel(logits, labels):
    lt = logits.T
    bt = labels.T
    partials = _ece_partials(lt, bt)
    ltail = logits[SC_ROWS:]
    btail = labels[SC_ROWS:]
    return _combine(partials, ltail, btail)
